# baseline (speedup 1.0000x reference)
import functools

import jax
import jax.numpy as jnp
from jax import lax
from jax.experimental import pallas as pl
from jax.experimental.pallas import tpu as pltpu
from jax.experimental.pallas import tpu_sc as plsc

NUM_USERS = 25000
NUM_ITEMS = 25000
N = NUM_USERS + NUM_ITEMS
E = 1600000
D = 32
N_LAYERS = 3

NC, NS = 2, 16
NT = NC * NS
N_PAD = 50176
R_T = N_PAD // NS
CHUNK = 440
T_E = E // NT
N_PAIRS = 56
TAIL = T_E - N_PAIRS * 2 * CHUNK - CHUNK
F = N_PAD * D // 128
TC_BLK = 896


def _sc_compiler_params():
  return pltpu.CompilerParams(use_tc_tiling_on_sc=False)


def _fill_rows(rows_ref, n_rows, value):
  vec = jnp.full((16,), value, jnp.float32)

  @pl.loop(0, n_rows)
  def _(r):
    rows_ref[r, pl.ds(0, 16)] = vec
    rows_ref[r, pl.ds(16, 16)] = vec


def _fill_rows16(rows_ref, n_rows, value):
  vec = jnp.full((32,), value, jnp.int16)

  @pl.loop(0, n_rows)
  def _(r):
    rows_ref[r, pl.ds(0, 32)] = vec


def _zero_acc_slab(rows_ref, acc, rbase, buf_rows):
  n_full = R_T // buf_rows
  rem = R_T - n_full * buf_rows

  @pl.loop(0, n_full)
  def _(i):
    pltpu.sync_copy(rows_ref, acc.at[pl.ds(rbase + i * buf_rows, buf_rows)])

  if rem:
    pltpu.sync_copy(rows_ref.at[pl.ds(0, rem)],
                    acc.at[pl.ds(rbase + n_full * buf_rows, rem)])


def _dump_slab(acc, rbase, c, out0, out1):
  @pl.when(c == 0)
  def _():
    pltpu.sync_copy(acc.at[pl.ds(rbase, R_T)], out0.at[pl.ds(rbase, R_T)])

  @pl.when(c == 1)
  def _():
    pltpu.sync_copy(acc.at[pl.ds(rbase, R_T)], out1.at[pl.ds(rbase, R_T)])


@functools.cache
def _make_sc_segment_sum():
  mesh = plsc.VectorSubcoreMesh(
      core_axis_name="c", subcore_axis_name="s", num_cores=NC, num_subcores=NS)
  part = jax.ShapeDtypeStruct((N_PAD, D), jnp.float32)

  @functools.partial(
      pl.kernel,
      out_type=[part, part],
      mesh=mesh,
      scratch_types=[
          pltpu.VMEM((2, CHUNK), jnp.int32),
          pltpu.VMEM((2, CHUNK), jnp.int32),
          pltpu.VMEM((2, TAIL), jnp.int32),
          pltpu.VMEM((CHUNK, D), jnp.float32),
          pltpu.VMEM((CHUNK, D), jnp.float32),
          pltpu.VMEM_SHARED((N_PAD, D), jnp.float32),
          pltpu.SemaphoreType.DMA,
          pltpu.SemaphoreType.DMA,
          pltpu.SemaphoreType.DMA,
      ],
      compiler_params=_sc_compiler_params(),
  )
  def sc_segment_sum(z_hbm, ei_hbm, out0, out1,
                     idx_a, idx_b, idx_t,
                     rows_a, rows_b, acc, gsem, ssem_a, ssem_b):
    c = lax.axis_index("c")
    s = lax.axis_index("s")
    wid = s * NC + c
    rbase = s * R_T
    _fill_rows(rows_a, CHUNK, 0.0)
    _zero_acc_slab(rows_a, acc, rbase, CHUNK)
    plsc.subcore_barrier()

    ebase = wid * T_E

    def scatter_desc(rows, idx, ssem):
      return pltpu.make_async_copy(rows, acc.at[idx.at[1]], ssem)

    def load_idx(eoff, n, idx):
      pltpu.sync_copy(ei_hbm.at[pl.ds(0, 2), pl.ds(eoff, n)], idx)

    def gather_split(idx, rows):
      gs = []
      for off, n in ((0, 224), (224, 216)):
        gs.append(pltpu.async_copy(z_hbm.at[idx.at[0, pl.ds(off, n)]],
                                   rows.at[pl.ds(off, n)], gsem))
      return gs

    def pair(p, first):
      eoff = ebase + p * 2 * CHUNK
      if not first:
        scatter_desc(rows_a, idx_a, ssem_a).wait()
      load_idx(eoff, CHUNK, idx_a)
      ga = gather_split(idx_a, rows_a)
      if not first:
        scatter_desc(rows_b, idx_b, ssem_b).wait()
      load_idx(eoff + CHUNK, CHUNK, idx_b)
      gb = gather_split(idx_b, rows_b)
      for g in ga:
        g.wait()
      pltpu.async_copy(rows_a, acc.at[idx_a.at[1]], ssem_a, add=True)
      for g in gb:
        g.wait()
      pltpu.async_copy(rows_b, acc.at[idx_b.at[1]], ssem_b, add=True)

    pair(0, True)

    @pl.loop(1, N_PAIRS)
    def _(p):
      pair(p, False)

    eoff = ebase + N_PAIRS * 2 * CHUNK
    scatter_desc(rows_a, idx_a, ssem_a).wait()
    load_idx(eoff, CHUNK, idx_a)
    ga = gather_split(idx_a, rows_a)
    scatter_desc(rows_b, idx_b, ssem_b).wait()
    load_idx(eoff + CHUNK, TAIL, idx_t)
    rows_t = rows_b.at[pl.ds(0, TAIL)]
    gb = pltpu.async_copy(z_hbm.at[idx_t.at[0]], rows_t, gsem)
    for g in ga:
      g.wait()
    pltpu.async_copy(rows_a, acc.at[idx_a.at[1]], ssem_a, add=True)
    gb.wait()
    pltpu.async_copy(rows_t, acc.at[idx_t.at[1]], ssem_b, add=True)
    scatter_desc(rows_a, idx_a, ssem_a).wait()
    pltpu.make_async_copy(rows_t, acc.at[idx_t.at[1]], ssem_b).wait()

    plsc.subcore_barrier()
    _dump_slab(acc, rbase, c, out0, out1)

  return sc_segment_sum


@functools.cache
def _make_sc_degree():
  mesh = plsc.VectorSubcoreMesh(
      core_axis_name="c", subcore_axis_name="s", num_cores=NC, num_subcores=NS)
  part = jax.ShapeDtypeStruct((N_PAD, D), jnp.int16)

  @functools.partial(
      pl.kernel,
      out_type=[part, part],
      mesh=mesh,
      scratch_types=[
          [pltpu.VMEM((CHUNK,), jnp.int32)] * 4,
          pltpu.VMEM((TAIL,), jnp.int32),
          pltpu.VMEM((CHUNK, D), jnp.int16),
          pltpu.VMEM_SHARED((N_PAD, D), jnp.int16),
          [pltpu.SemaphoreType.DMA] * 4,
      ],
      compiler_params=_sc_compiler_params(),
  )
  def sc_degree(ei_hbm, out0, out1, didxs, didx_t, ones_rows, acc, ssems):
    c = lax.axis_index("c")
    s = lax.axis_index("s")
    wid = s * NC + c
    rbase = s * R_T
    _fill_rows16(ones_rows, CHUNK, 0)
    _zero_acc_slab(ones_rows, acc, rbase, CHUNK)
    _fill_rows16(ones_rows, CHUNK, 1)
    plsc.subcore_barrier()

    ebase = wid * T_E
    NQ = 4
    N_QUADS = T_E // (NQ * CHUNK)
    REST = T_E - N_QUADS * NQ * CHUNK - TAIL

    def half(eoff, didx, ssem, first):
      if not first:
        pltpu.make_async_copy(ones_rows, acc.at[didx], ssem).wait()
      pltpu.sync_copy(ei_hbm.at[1, pl.ds(eoff, CHUNK)], didx)
      pltpu.async_copy(ones_rows, acc.at[didx], ssem, add=True)

    def quad(q, first):
      eoff = ebase + q * NQ * CHUNK
      for j in range(NQ):
        half(eoff + j * CHUNK, didxs[j], ssems[j], first)

    quad(0, True)

    @pl.loop(1, N_QUADS)
    def _(q):
      quad(q, False)

    eoff = ebase + N_QUADS * NQ * CHUNK
    n_rest = REST // CHUNK
    for j in range(n_rest):
      half(eoff + j * CHUNK, didxs[j], ssems[j], False)
    pltpu.sync_copy(ei_hbm.at[1, pl.ds(eoff + n_rest * CHUNK, TAIL)], didx_t)
    ones_t = ones_rows.at[pl.ds(0, TAIL)]
    pltpu.make_async_copy(ones_rows, acc.at[didxs[n_rest]],
                          ssems[n_rest]).wait()
    pltpu.async_copy(ones_t, acc.at[didx_t], ssems[n_rest], add=True)

    for j in range(NQ):
      if j == n_rest:
        pltpu.make_async_copy(ones_t, acc.at[didx_t], ssems[j]).wait()
      else:
        pltpu.make_async_copy(ones_rows, acc.at[didxs[j]], ssems[j]).wait()

    plsc.subcore_barrier()
    _dump_slab(acc, rbase, c, out0, out1)

  return sc_degree


def _tc_call(body, n_in, n_out):
    spec = pl.BlockSpec((TC_BLK, 128), lambda i: (i, 0))
    return pl.pallas_call(
        body,
        grid=(F // TC_BLK,),
        in_specs=[spec] * n_in,
        out_specs=[spec] * n_out if n_out > 1 else spec,
        out_shape=(
            [jax.ShapeDtypeStruct((F, 128), jnp.float32)] * n_out
            if n_out > 1 else jax.ShapeDtypeStruct((F, 128), jnp.float32)),
    )


def _tc_init_body(d0_ref, d1_ref, x0_ref, dinv_ref, z0_ref):
    deg = (d0_ref[...].astype(jnp.float32) +
           d1_ref[...].astype(jnp.float32))
    dinv = jnp.where(deg > 0, lax.rsqrt(jnp.maximum(deg, 1e-12)),
                     jnp.float32(0.0))
    dinv_ref[...] = dinv
    z0_ref[...] = x0_ref[...] * dinv


def _tc_combine_body(p0_ref, p1_ref, dinv_ref, x_ref, z_ref):
    dinv = dinv_ref[...]
    x = dinv * (p0_ref[...] + p1_ref[...])
    x_ref[...] = x
    z_ref[...] = x * dinv


def _tc_final_body(p0_ref, p1_ref, dinv_ref, x0_ref, x1_ref, x2_ref, out_ref):
    x3 = dinv_ref[...] * (p0_ref[...] + p1_ref[...])
    out_ref[...] = (x0_ref[...] + x1_ref[...] + x2_ref[...] + x3) * 0.25


def _flat(a):
    return a.reshape(F, 128)


def kernel(user_emb, item_emb, edge_index):
    ei = edge_index.astype(jnp.int32)

    fu = NUM_USERS * D // 128
    x0f = jnp.concatenate(
        [user_emb.reshape(fu, 128), item_emb.reshape(fu, 128),
         jnp.zeros((F - 2 * fu, 128), jnp.float32)], axis=0)

    sc_segment_sum = _make_sc_segment_sum()
    dg0, dg1 = _make_sc_degree()(ei)
    dinvf, zf = _tc_call(_tc_init_body, 3, 2)(_flat(dg0), _flat(dg1), x0f)

    xfs = []
    for _ in range(N_LAYERS - 1):
        p0, p1 = sc_segment_sum(zf.reshape(N_PAD, D), ei)
        xf, zf = _tc_call(_tc_combine_body, 3, 2)(_flat(p0), _flat(p1), dinvf)
        xfs.append(xf)
    p0, p1 = sc_segment_sum(zf.reshape(N_PAD, D), ei)
    outf = _tc_call(_tc_final_body, 6, 1)(
        _flat(p0), _flat(p1), dinvf, x0f, xfs[0], xfs[1])

    fu = NUM_USERS * D // 128
    users = outf[:fu].reshape(NUM_USERS, D)
    items = outf[fu:2 * fu].reshape(NUM_ITEMS, D)
    return (users, items)


# speedup vs baseline: 1.0297x; 1.0297x over previous
import functools

import jax
import jax.numpy as jnp
from jax import lax
from jax.experimental import pallas as pl
from jax.experimental.pallas import tpu as pltpu
from jax.experimental.pallas import tpu_sc as plsc

NUM_USERS = 25000
NUM_ITEMS = 25000
N = NUM_USERS + NUM_ITEMS
E = 1600000
D = 32
N_LAYERS = 3

NC, NS = 2, 16
NT = NC * NS
N_PAD = 50176
R_T = N_PAD // NS
CHUNK = 440
T_E = E // NT
N_PAIRS = 56
TAIL = T_E - N_PAIRS * 2 * CHUNK - CHUNK
F = N_PAD * D // 128
TC_BLK = 896


def _sc_compiler_params():
  return pltpu.CompilerParams(use_tc_tiling_on_sc=False)


def _fill_rows(rows_ref, n_rows, value):
  vec = jnp.full((16,), value, jnp.float32)

  @pl.loop(0, n_rows)
  def _(r):
    rows_ref[r, pl.ds(0, 16)] = vec
    rows_ref[r, pl.ds(16, 16)] = vec


def _zero_acc_slab(rows_ref, acc, rbase, buf_rows):
  n_full = R_T // buf_rows
  rem = R_T - n_full * buf_rows

  @pl.loop(0, n_full)
  def _(i):
    pltpu.sync_copy(rows_ref, acc.at[pl.ds(rbase + i * buf_rows, buf_rows)])

  if rem:
    pltpu.sync_copy(rows_ref.at[pl.ds(0, rem)],
                    acc.at[pl.ds(rbase + n_full * buf_rows, rem)])


def _dump_slab(acc, rbase, c, out0, out1):
  @pl.when(c == 0)
  def _():
    pltpu.sync_copy(acc.at[pl.ds(rbase, R_T)], out0.at[pl.ds(rbase, R_T)])

  @pl.when(c == 1)
  def _():
    pltpu.sync_copy(acc.at[pl.ds(rbase, R_T)], out1.at[pl.ds(rbase, R_T)])


@functools.cache
def _make_sc_segment_sum():
  mesh = plsc.VectorSubcoreMesh(
      core_axis_name="c", subcore_axis_name="s", num_cores=NC, num_subcores=NS)
  part = jax.ShapeDtypeStruct((N_PAD, D), jnp.float32)

  @functools.partial(
      pl.kernel,
      out_type=[part, part],
      mesh=mesh,
      scratch_types=[
          pltpu.VMEM((2, CHUNK), jnp.int32),
          pltpu.VMEM((2, CHUNK), jnp.int32),
          pltpu.VMEM((2, TAIL), jnp.int32),
          pltpu.VMEM((CHUNK, D), jnp.float32),
          pltpu.VMEM((CHUNK, D), jnp.float32),
          pltpu.VMEM_SHARED((N_PAD, D), jnp.float32),
          pltpu.SemaphoreType.DMA,
          pltpu.SemaphoreType.DMA,
          pltpu.SemaphoreType.DMA,
      ],
      compiler_params=_sc_compiler_params(),
  )
  def sc_segment_sum(z_hbm, ei_hbm, out0, out1,
                     idx_a, idx_b, idx_t,
                     rows_a, rows_b, acc, gsem, ssem_a, ssem_b):
    c = lax.axis_index("c")
    s = lax.axis_index("s")
    wid = s * NC + c
    rbase = s * R_T
    _fill_rows(rows_a, CHUNK, 0.0)
    _zero_acc_slab(rows_a, acc, rbase, CHUNK)
    plsc.subcore_barrier()

    ebase = wid * T_E

    def scatter_desc(rows, idx, ssem):
      return pltpu.make_async_copy(rows, acc.at[idx.at[1]], ssem)

    def load_idx(eoff, n, idx):
      pltpu.sync_copy(ei_hbm.at[pl.ds(0, 2), pl.ds(eoff, n)], idx)

    def gather_split(idx, rows):
      gs = []
      for off, n in ((0, 224), (224, 216)):
        gs.append(pltpu.async_copy(z_hbm.at[idx.at[0, pl.ds(off, n)]],
                                   rows.at[pl.ds(off, n)], gsem))
      return gs

    def pair(p, first):
      eoff = ebase + p * 2 * CHUNK
      if not first:
        scatter_desc(rows_a, idx_a, ssem_a).wait()
      load_idx(eoff, CHUNK, idx_a)
      ga = gather_split(idx_a, rows_a)
      if not first:
        scatter_desc(rows_b, idx_b, ssem_b).wait()
      load_idx(eoff + CHUNK, CHUNK, idx_b)
      gb = gather_split(idx_b, rows_b)
      for g in ga:
        g.wait()
      pltpu.async_copy(rows_a, acc.at[idx_a.at[1]], ssem_a, add=True)
      for g in gb:
        g.wait()
      pltpu.async_copy(rows_b, acc.at[idx_b.at[1]], ssem_b, add=True)

    pair(0, True)

    @pl.loop(1, N_PAIRS)
    def _(p):
      pair(p, False)

    eoff = ebase + N_PAIRS * 2 * CHUNK
    scatter_desc(rows_a, idx_a, ssem_a).wait()
    load_idx(eoff, CHUNK, idx_a)
    ga = gather_split(idx_a, rows_a)
    scatter_desc(rows_b, idx_b, ssem_b).wait()
    load_idx(eoff + CHUNK, TAIL, idx_t)
    rows_t = rows_b.at[pl.ds(0, TAIL)]
    gb = pltpu.async_copy(z_hbm.at[idx_t.at[0]], rows_t, gsem)
    for g in ga:
      g.wait()
    pltpu.async_copy(rows_a, acc.at[idx_a.at[1]], ssem_a, add=True)
    gb.wait()
    pltpu.async_copy(rows_t, acc.at[idx_t.at[1]], ssem_b, add=True)
    scatter_desc(rows_a, idx_a, ssem_a).wait()
    pltpu.make_async_copy(rows_t, acc.at[idx_t.at[1]], ssem_b).wait()

    plsc.subcore_barrier()
    _dump_slab(acc, rbase, c, out0, out1)

  return sc_segment_sum


@functools.cache
def _make_sc_degree():
  mesh = plsc.VectorSubcoreMesh(
      core_axis_name="c", subcore_axis_name="s", num_cores=NC, num_subcores=NS)
  part = jax.ShapeDtypeStruct((N_PAD, D), jnp.float32)

  @functools.partial(
      pl.kernel,
      out_type=[part, part],
      mesh=mesh,
      scratch_types=[
          [pltpu.VMEM((CHUNK,), jnp.int32)] * 4,
          pltpu.VMEM((TAIL,), jnp.int32),
          pltpu.VMEM((CHUNK, D), jnp.float32),
          pltpu.VMEM_SHARED((N_PAD, D), jnp.float32),
          [pltpu.SemaphoreType.DMA] * 4,
      ],
      compiler_params=_sc_compiler_params(),
  )
  def sc_degree(ei_hbm, out0, out1, didxs, didx_t, ones_rows, acc, ssems):
    c = lax.axis_index("c")
    s = lax.axis_index("s")
    wid = s * NC + c
    rbase = s * R_T
    _fill_rows(ones_rows, CHUNK, 0.0)
    _zero_acc_slab(ones_rows, acc, rbase, CHUNK)
    _fill_rows(ones_rows, CHUNK, 1.0)
    plsc.subcore_barrier()

    ebase = wid * T_E
    NQ = 4
    N_QUADS = T_E // (NQ * CHUNK)
    REST = T_E - N_QUADS * NQ * CHUNK - TAIL

    def half(eoff, didx, ssem, first):
      if not first:
        pltpu.make_async_copy(ones_rows, acc.at[didx], ssem).wait()
      pltpu.sync_copy(ei_hbm.at[1, pl.ds(eoff, CHUNK)], didx)
      pltpu.async_copy(ones_rows, acc.at[didx], ssem, add=True)

    def quad(q, first):
      eoff = ebase + q * NQ * CHUNK
      for j in range(NQ):
        half(eoff + j * CHUNK, didxs[j], ssems[j], first)

    quad(0, True)

    @pl.loop(1, N_QUADS)
    def _(q):
      quad(q, False)

    eoff = ebase + N_QUADS * NQ * CHUNK
    n_rest = REST // CHUNK
    for j in range(n_rest):
      half(eoff + j * CHUNK, didxs[j], ssems[j], False)
    pltpu.sync_copy(ei_hbm.at[1, pl.ds(eoff + n_rest * CHUNK, TAIL)], didx_t)
    ones_t = ones_rows.at[pl.ds(0, TAIL)]
    pltpu.make_async_copy(ones_rows, acc.at[didxs[n_rest]],
                          ssems[n_rest]).wait()
    pltpu.async_copy(ones_t, acc.at[didx_t], ssems[n_rest], add=True)

    for j in range(NQ):
      if j == n_rest:
        pltpu.make_async_copy(ones_t, acc.at[didx_t], ssems[j]).wait()
      else:
        pltpu.make_async_copy(ones_rows, acc.at[didxs[j]], ssems[j]).wait()

    plsc.subcore_barrier()
    _dump_slab(acc, rbase, c, out0, out1)

  return sc_degree


def _tc_call(body, n_in, n_out):
    spec = pl.BlockSpec((TC_BLK, 128), lambda i: (i, 0))
    return pl.pallas_call(
        body,
        grid=(F // TC_BLK,),
        in_specs=[spec] * n_in,
        out_specs=[spec] * n_out if n_out > 1 else spec,
        out_shape=(
            [jax.ShapeDtypeStruct((F, 128), jnp.float32)] * n_out
            if n_out > 1 else jax.ShapeDtypeStruct((F, 128), jnp.float32)),
    )


def _tc_init_body(d0_ref, d1_ref, x0_ref, dinv_ref, z0_ref):
    deg = d0_ref[...] + d1_ref[...]
    dinv = jnp.where(deg > 0, lax.rsqrt(jnp.maximum(deg, 1e-12)),
                     jnp.float32(0.0))
    dinv_ref[...] = dinv
    z0_ref[...] = x0_ref[...] * dinv


def _tc_combine_body(p0_ref, p1_ref, dinv_ref, x_ref, z_ref):
    dinv = dinv_ref[...]
    x = dinv * (p0_ref[...] + p1_ref[...])
    x_ref[...] = x
    z_ref[...] = x * dinv


def _tc_final_body(p0_ref, p1_ref, dinv_ref, x0_ref, x1_ref, x2_ref, out_ref):
    x3 = dinv_ref[...] * (p0_ref[...] + p1_ref[...])
    out_ref[...] = (x0_ref[...] + x1_ref[...] + x2_ref[...] + x3) * 0.25


def _flat(a):
    return a.reshape(F, 128)


def kernel(user_emb, item_emb, edge_index):
    ei = edge_index.astype(jnp.int32)

    fu = NUM_USERS * D // 128
    x0f = jnp.concatenate(
        [user_emb.reshape(fu, 128), item_emb.reshape(fu, 128),
         jnp.zeros((F - 2 * fu, 128), jnp.float32)], axis=0)

    sc_segment_sum = _make_sc_segment_sum()
    dg0, dg1 = _make_sc_degree()(ei)
    dinvf, zf = _tc_call(_tc_init_body, 3, 2)(_flat(dg0), _flat(dg1), x0f)

    xfs = []
    for _ in range(N_LAYERS - 1):
        p0, p1 = sc_segment_sum(zf.reshape(N_PAD, D), ei)
        xf, zf = _tc_call(_tc_combine_body, 3, 2)(_flat(p0), _flat(p1), dinvf)
        xfs.append(xf)
    p0, p1 = sc_segment_sum(zf.reshape(N_PAD, D), ei)
    outf = _tc_call(_tc_final_body, 6, 1)(
        _flat(p0), _flat(p1), dinvf, x0f, xfs[0], xfs[1])

    fu = NUM_USERS * D // 128
    users = outf[:fu].reshape(NUM_USERS, D)
    items = outf[fu:2 * fu].reshape(NUM_ITEMS, D)
    return (users, items)
